# baseline (speedup 1.0000x reference)
import functools
import math

import jax
import jax.numpy as jnp
from jax import lax
from jax.experimental import pallas as pl
from jax.experimental.pallas import tpu as pltpu
from jax.experimental.pallas import tpu_sc as plsc

N_EMBD = 128
SCALE = math.sqrt(N_EMBD)

NC, NS, L = 2, 16, 16
NW = NC * NS

NSENTS = 4096
SLEN = 50
S_PER_W = NSENTS // NW
NPAIR = SLEN // 2
NBUF = 3
LA = 1


@jax.jit
def _sc_embed(tokens_g, table):
    mesh = plsc.VectorSubcoreMesh(core_axis_name="c", subcore_axis_name="s")

    @functools.partial(
        pl.kernel,
        out_type=jax.ShapeDtypeStruct((SLEN, NSENTS, N_EMBD), jnp.float32),
        mesh=mesh,
        scratch_types=[
            pltpu.VMEM((SLEN, S_PER_W), jnp.int32),
            [pltpu.VMEM((2, S_PER_W, N_EMBD), jnp.float32)
             for _ in range(NBUF)],
            [pltpu.SemaphoreType.DMA for _ in range(NBUF)],
            [pltpu.SemaphoreType.DMA for _ in range(NBUF)],
        ],
    )
    def k(idx_hbm, table_hbm, out_hbm, idx_v, bufs, gsems, ssems):
        wid = lax.axis_index("s") * NC + lax.axis_index("c")
        i0 = wid * S_PER_W
        pltpu.sync_copy(idx_hbm.at[:, pl.ds(i0, S_PER_W)], idx_v)

        def gather(p):
            b = p % NBUF
            return [
                pltpu.async_copy(
                    table_hbm.at[idx_v.at[2 * p + q]], bufs[b].at[q],
                    gsems[b])
                for q in range(2)
            ]

        def store(p):
            b = p % NBUF
            return pltpu.async_copy(
                bufs[b],
                out_hbm.at[pl.ds(2 * p, 2), pl.ds(i0, S_PER_W)], ssems[b])

        def scale(p):
            buf = bufs[p % NBUF]

            def row_body(r, carry):
                for q in range(2):
                    for c in range(N_EMBD // L):
                        sl = pl.ds(c * L, L)
                        buf[q, r, sl] = buf[q, r, sl] * SCALE
                return carry

            lax.fori_loop(0, S_PER_W, row_body, 0)

        gathers, stores = {}, {}
        for p in range(LA):
            gathers[p] = gather(p)

        for p in range(NPAIR):
            t = p + LA
            if t < NPAIR:
                if t - NBUF >= 0:
                    stores.pop(t - NBUF).wait()
                gathers[t] = gather(t)
            for h in gathers.pop(p):
                h.wait()
            scale(p)
            stores[p] = store(p)

        for p in sorted(stores):
            stores.pop(p).wait()

    return k(tokens_g, table)


def kernel(tokens, table):
    tokens_t = tokens.astype(jnp.int32).T
    out = _sc_embed(tokens_t, table)
    return out.transpose(1, 0, 2)


# speedup vs baseline: 1.0122x; 1.0122x over previous
import functools
import math

import jax
import jax.numpy as jnp
from jax import lax
from jax.experimental import pallas as pl
from jax.experimental.pallas import tpu as pltpu
from jax.experimental.pallas import tpu_sc as plsc

N_EMBD = 128
SCALE = math.sqrt(N_EMBD)

NC, NS, L = 2, 16, 16
NW = NC * NS

NSENTS = 4096
SLEN = 50
S_PER_W = NSENTS // NW
NBUF = 6
LA = 2


@jax.jit
def _sc_embed(tokens_g, table):
    mesh = plsc.VectorSubcoreMesh(core_axis_name="c", subcore_axis_name="s")

    @functools.partial(
        pl.kernel,
        out_type=jax.ShapeDtypeStruct((SLEN, NSENTS, N_EMBD), jnp.float32),
        mesh=mesh,
        scratch_types=[
            pltpu.VMEM((SLEN, S_PER_W), jnp.int32),
            [pltpu.VMEM((S_PER_W, N_EMBD), jnp.float32) for _ in range(NBUF)],
            [pltpu.SemaphoreType.DMA for _ in range(NBUF)],
            [pltpu.SemaphoreType.DMA for _ in range(NBUF)],
        ],
    )
    def k(idx_hbm, table_hbm, out_hbm, idx_v, bufs, gsems, ssems):
        wid = lax.axis_index("s") * NC + lax.axis_index("c")
        i0 = wid * S_PER_W
        pltpu.sync_copy(idx_hbm.at[:, pl.ds(i0, S_PER_W)], idx_v)

        def gather(j):
            b = j % NBUF
            return pltpu.async_copy(
                table_hbm.at[idx_v.at[j]], bufs[b], gsems[b])

        def store(j):
            b = j % NBUF
            return pltpu.async_copy(
                bufs[b], out_hbm.at[j, pl.ds(i0, S_PER_W)], ssems[b])

        def scale(j):
            buf = bufs[j % NBUF]

            def row_body(r, carry):
                for c in range(N_EMBD // L):
                    sl = pl.ds(c * L, L)
                    buf[r, sl] = buf[r, sl] * SCALE
                return carry

            lax.fori_loop(0, S_PER_W, row_body, 0)

        gathers, stores = {}, {}
        for j in range(LA):
            gathers[j] = gather(j)

        for j in range(SLEN):
            t = j + LA
            if t < SLEN:
                if t - NBUF >= 0:
                    stores.pop(t - NBUF).wait()
                gathers[t] = gather(t)
            gathers.pop(j).wait()
            scale(j)
            stores[j] = store(j)

        for j in sorted(stores):
            stores.pop(j).wait()

    return k(tokens_g, table)


def kernel(tokens, table):
    tokens_t = tokens.astype(jnp.int32).T
    out = _sc_embed(tokens_t, table)
    return out.transpose(1, 0, 2)


# speedup vs baseline: 1.0149x; 1.0027x over previous
import functools
import math

import jax
import jax.numpy as jnp
from jax import lax
from jax.experimental import pallas as pl
from jax.experimental.pallas import tpu as pltpu
from jax.experimental.pallas import tpu_sc as plsc

N_EMBD = 128
SCALE = math.sqrt(N_EMBD)

NC, NS, L = 2, 16, 16
NW = NC * NS

NSENTS = 4096
SLEN = 50
S_PER_W = NSENTS // NW
NBUF = 6
LA = 3


@jax.jit
def _sc_embed(tokens_g, table):
    mesh = plsc.VectorSubcoreMesh(core_axis_name="c", subcore_axis_name="s")

    @functools.partial(
        pl.kernel,
        out_type=jax.ShapeDtypeStruct((SLEN, NSENTS, N_EMBD), jnp.float32),
        mesh=mesh,
        scratch_types=[
            pltpu.VMEM((SLEN, S_PER_W), jnp.int32),
            [pltpu.VMEM((S_PER_W, N_EMBD), jnp.float32) for _ in range(NBUF)],
            [pltpu.SemaphoreType.DMA for _ in range(NBUF)],
            [pltpu.SemaphoreType.DMA for _ in range(NBUF)],
        ],
    )
    def k(idx_hbm, table_hbm, out_hbm, idx_v, bufs, gsems, ssems):
        wid = lax.axis_index("s") * NC + lax.axis_index("c")
        i0 = wid * S_PER_W
        pltpu.sync_copy(idx_hbm.at[:, pl.ds(i0, S_PER_W)], idx_v)

        def gather(j):
            b = j % NBUF
            return pltpu.async_copy(
                table_hbm.at[idx_v.at[j]], bufs[b], gsems[b])

        def store(j):
            b = j % NBUF
            return pltpu.async_copy(
                bufs[b], out_hbm.at[j, pl.ds(i0, S_PER_W)], ssems[b])

        def scale(j):
            buf = bufs[j % NBUF]

            def row_body(r, carry):
                for c in range(N_EMBD // L):
                    sl = pl.ds(c * L, L)
                    buf[r, sl] = buf[r, sl] * SCALE
                return carry

            lax.fori_loop(0, S_PER_W, row_body, 0)

        gathers, stores = {}, {}
        for j in range(LA):
            gathers[j] = gather(j)

        for j in range(SLEN):
            t = j + LA
            if t < SLEN:
                if t - NBUF >= 0:
                    stores.pop(t - NBUF).wait()
                gathers[t] = gather(t)
            gathers.pop(j).wait()
            scale(j)
            stores[j] = store(j)

        for j in sorted(stores):
            stores.pop(j).wait()

    return k(tokens_g, table)


def kernel(tokens, table):
    tokens_t = tokens.astype(jnp.int32).T
    out = _sc_embed(tokens_t, table)
    return out.transpose(1, 0, 2)
